# baseline (speedup 1.0000x reference)
import functools

import jax
import jax.numpy as jnp
from jax import lax
from jax.experimental import pallas as pl
from jax.experimental.pallas import tpu as pltpu
from jax.experimental.pallas import tpu_sc as plsc

_L = 16


def _al(x, n=16):
    return pl.multiple_of(x, n)


def _row_argmax(score_ref, buf, T):
    lane_iota = lax.iota(jnp.int32, _L)
    neg_inf = jnp.full((_L,), -jnp.inf, dtype=jnp.float32)
    half_t = T // 256

    def body(t, carry):
        maxa, besta, curia, maxb, bestb, curib = carry
        for k in range(128 // _L):
            xa = score_ref[buf, t, pl.ds(k * _L, _L)]
            xb = score_ref[buf, half_t + t, pl.ds(k * _L, _L)]
            ga = xa > maxa
            gb = xb > maxb
            maxa = jnp.where(ga, xa, maxa)
            besta = jnp.where(ga, curia, besta)
            maxb = jnp.where(gb, xb, maxb)
            bestb = jnp.where(gb, curib, bestb)
            curia = curia + _L
            curib = curib + _L
        return maxa, besta, curia, maxb, bestb, curib

    ib0 = lane_iota + T // 2
    maxa, besta, _, maxb, bestb, _ = lax.fori_loop(
        0, half_t, body, (neg_inf, lane_iota, lane_iota, neg_inf, ib0, ib0),
        unroll=1,
    )
    gb = maxb > maxa
    maxv = jnp.where(gb, maxb, maxa)
    besti = jnp.where(gb, bestb, besta)
    m = jnp.max(maxv)
    masked = jnp.where(maxv == m, besti, jnp.int32(2147483647))
    return jnp.min(masked)


def _lane_extract(vec, lane):
    sel = lax.iota(jnp.int32, _L) == lane
    return jnp.sum(jnp.where(sel, vec, jnp.zeros_like(vec)))


def _sc_kernel_body(NS, B, T, ML, RPW,
                    score_hbm, off_hbm, frames_hbm, out_hbm,
                    sbuf, obuf, fbuf, outbuf, shared,
                    sems, osems, sem_f, sem_m):
    cid = lax.axis_index("c")
    sid = lax.axis_index("s")
    wid = cid * NS + sid
    base = wid * RPW
    g = base // 8

    copies = [
        pltpu.async_copy(score_hbm.at[g, :, base % 8 + r], sbuf.at[r],
                         sems[r])
        for r in range(RPW)
    ]
    inds, ocopies = [], []
    for r in range(RPW):
        copies[r].wait()
        ind = _row_argmax(sbuf, r, T)
        inds.append(ind)
        for comp in range(2):
            ocopies.append(pltpu.async_copy(
                off_hbm.at[base + r, ind >> 7, comp,
                           pl.ds(_al((ind & 127) & ~15), _L)],
                obuf.at[2 * r + comp], osems[r],
            ))

    s_idx, e_idx, fcopies = [], [], []
    for r in range(RPW):
        ocopies[2 * r].wait()
        ocopies[2 * r + 1].wait()
        ind = inds[r]
        lane = ind & 15
        off0 = _lane_extract(obuf[2 * r], lane)
        off1 = _lane_extract(obuf[2 * r + 1], lane)
        center = (ind >> 2).astype(jnp.float32)
        half = (jnp.int32(4) << (ind & 3)).astype(jnp.float32)
        sf = (center - half) + off0
        ef = (center + half) + off1
        s = sf.astype(jnp.int32)
        s = s - (s.astype(jnp.float32) > sf).astype(jnp.int32)
        e = ef.astype(jnp.int32)
        e = e - (e.astype(jnp.float32) > ef).astype(jnp.int32)
        s_idx.append(jnp.clip(s, 0, ML - 1))
        e_idx.append(jnp.clip(e, 0, ML - 1))
        for k, v in ((r, s_idx[r]), (RPW + r, e_idx[r])):
            fcopies.append(pltpu.async_copy(
                frames_hbm.at[g, v >> 7, base % 8 + r,
                              pl.ds(_al((v & 127) & ~15), _L)],
                fbuf.at[k], sem_f,
            ))
    for c in fcopies:
        c.wait()

    lane_iota = lax.iota(jnp.int32, _L)
    acc = jnp.zeros((_L,), dtype=jnp.float32)
    for r in range(RPW):
        f0 = _lane_extract(fbuf[r].astype(jnp.float32), s_idx[r] & 15)
        f1 = _lane_extract(fbuf[RPW + r].astype(jnp.float32), e_idx[r] & 15)
        acc = jnp.where(lane_iota == base % 8 + r, f0, acc)
        acc = jnp.where(lane_iota == 8 + base % 8 + r, f1 + 1.0, acc)

    @pl.when(sid % 2 == 1)
    def _():
        outbuf[...] = acc
        pltpu.sync_copy(outbuf, shared.at[sid])

    plsc.subcore_barrier()

    @pl.when(sid % 2 == 0)
    def _():
        pltpu.async_copy(shared.at[sid + 1], outbuf, sem_m).wait()
        outbuf[...] = acc + outbuf[...]
        b0 = base
        c1 = pltpu.async_copy(outbuf.at[pl.ds(0, 8)],
                              out_hbm.at[pl.ds(_al(b0, 8), 8)], sem_m)
        c2 = pltpu.async_copy(outbuf.at[pl.ds(8, 8)],
                              out_hbm.at[pl.ds(_al(B + b0, 8), 8)], sem_m)
        c1.wait()
        c2.wait()


def kernel(temporal_score, temporal_offset, frames_id):
    B, T = temporal_score.shape
    ML = frames_id.shape[1]
    info = plsc.get_sparse_core_info()
    NC, NS = info.num_cores, info.num_subcores
    NW = NC * NS
    RPW = B // NW

    score_po = temporal_score.reshape(B // 8, 8, T // 128, 128).transpose(
        0, 2, 1, 3)
    off_po = temporal_offset.reshape(B, T // 128, 128, 2).transpose(
        0, 1, 3, 2)
    frames_po = frames_id.reshape(B // 8, 8, ML // 128, 128).transpose(
        0, 2, 1, 3)

    mesh = plsc.VectorSubcoreMesh(core_axis_name="c", subcore_axis_name="s")
    k = functools.partial(
        pl.kernel,
        mesh=mesh,
        out_type=jax.ShapeDtypeStruct((2 * B,), jnp.float32),
        scratch_types=[
            pltpu.VMEM((RPW, T // 128, 128), jnp.float32),
            pltpu.VMEM((2 * RPW, _L), jnp.float32),
            pltpu.VMEM((2 * RPW, _L), jnp.int32),
            pltpu.VMEM((_L,), jnp.float32),
            pltpu.VMEM_SHARED((NS, _L), jnp.float32),
            [pltpu.SemaphoreType.DMA] * RPW,
            [pltpu.SemaphoreType.DMA] * RPW,
            pltpu.SemaphoreType.DMA,
            pltpu.SemaphoreType.DMA,
        ],
        compiler_params=pltpu.CompilerParams(
            use_tc_tiling_on_sc=False, needs_layout_passes=False
        ),
    )(functools.partial(_sc_kernel_body, NS, B, T, ML, RPW))
    return k(score_po, off_po, frames_po).reshape(2, B).T


# speedup vs baseline: 1.0122x; 1.0122x over previous
import functools

import jax
import jax.numpy as jnp
from jax import lax
from jax.experimental import pallas as pl
from jax.experimental.pallas import tpu as pltpu
from jax.experimental.pallas import tpu_sc as plsc

_L = 16


def _al(x, n=16):
    return pl.multiple_of(x, n)


def _row_argmax(score_ref, buf, T):
    lane_iota = lax.iota(jnp.int32, _L)
    neg_inf = jnp.full((_L,), -jnp.inf, dtype=jnp.float32)
    half_t = T // 256

    def body(t, carry):
        maxa, besta, curia, maxb, bestb, curib = carry
        for k in range(128 // _L):
            xa = score_ref[buf, t, pl.ds(k * _L, _L)]
            xb = score_ref[buf, half_t + t, pl.ds(k * _L, _L)]
            ga = xa > maxa
            gb = xb > maxb
            maxa = jnp.where(ga, xa, maxa)
            besta = jnp.where(ga, curia, besta)
            maxb = jnp.where(gb, xb, maxb)
            bestb = jnp.where(gb, curib, bestb)
            curia = curia + _L
            curib = curib + _L
        return maxa, besta, curia, maxb, bestb, curib

    ib0 = lane_iota + T // 2
    maxa, besta, _, maxb, bestb, _ = lax.fori_loop(
        0, half_t, body, (neg_inf, lane_iota, lane_iota, neg_inf, ib0, ib0),
        unroll=2,
    )
    gb = maxb > maxa
    maxv = jnp.where(gb, maxb, maxa)
    besti = jnp.where(gb, bestb, besta)
    m = jnp.max(maxv)
    masked = jnp.where(maxv == m, besti, jnp.int32(2147483647))
    return jnp.min(masked)


def _lane_extract(vec, lane):
    sel = lax.iota(jnp.int32, _L) == lane
    return jnp.sum(jnp.where(sel, vec, jnp.zeros_like(vec)))


def _sc_kernel_body(NS, B, T, ML, RPW,
                    score_hbm, off_hbm, frames_hbm, out_hbm,
                    sbuf, obuf, fbuf, outbuf, shared,
                    sems, osems, sem_f, sem_m):
    cid = lax.axis_index("c")
    sid = lax.axis_index("s")
    wid = cid * NS + sid
    base = wid * RPW
    g = base // 8

    def row_copy(r):
        return pltpu.async_copy(score_hbm.at[g, :, base % 8 + r],
                                sbuf.at[r], sems[r])

    copies = [row_copy(0), row_copy(1)]
    inds, ocopies = [], []
    for r in range(RPW):
        copies[r].wait()
        if r == 0:
            copies += [row_copy(rr) for rr in range(2, RPW)]
        ind = _row_argmax(sbuf, r, T)
        inds.append(ind)
        for comp in range(2):
            ocopies.append(pltpu.async_copy(
                off_hbm.at[base + r, ind >> 7, comp,
                           pl.ds(_al((ind & 127) & ~15), _L)],
                obuf.at[2 * r + comp], osems[r],
            ))

    s_idx, e_idx, fcopies = [], [], []
    for r in range(RPW):
        ocopies[2 * r].wait()
        ocopies[2 * r + 1].wait()
        ind = inds[r]
        lane = ind & 15
        off0 = _lane_extract(obuf[2 * r], lane)
        off1 = _lane_extract(obuf[2 * r + 1], lane)
        center = (ind >> 2).astype(jnp.float32)
        half = (jnp.int32(4) << (ind & 3)).astype(jnp.float32)
        sf = (center - half) + off0
        ef = (center + half) + off1
        s = sf.astype(jnp.int32)
        s = s - (s.astype(jnp.float32) > sf).astype(jnp.int32)
        e = ef.astype(jnp.int32)
        e = e - (e.astype(jnp.float32) > ef).astype(jnp.int32)
        s_idx.append(jnp.clip(s, 0, ML - 1))
        e_idx.append(jnp.clip(e, 0, ML - 1))
        for k, v in ((r, s_idx[r]), (RPW + r, e_idx[r])):
            fcopies.append(pltpu.async_copy(
                frames_hbm.at[g, v >> 7, base % 8 + r,
                              pl.ds(_al((v & 127) & ~15), _L)],
                fbuf.at[k], sem_f,
            ))
    for c in fcopies:
        c.wait()

    lane_iota = lax.iota(jnp.int32, _L)
    acc = jnp.zeros((_L,), dtype=jnp.float32)
    for r in range(RPW):
        f0 = _lane_extract(fbuf[r].astype(jnp.float32), s_idx[r] & 15)
        f1 = _lane_extract(fbuf[RPW + r].astype(jnp.float32), e_idx[r] & 15)
        acc = jnp.where(lane_iota == base % 8 + r, f0, acc)
        acc = jnp.where(lane_iota == 8 + base % 8 + r, f1 + 1.0, acc)

    @pl.when(sid % 2 == 1)
    def _():
        outbuf[...] = acc
        pltpu.sync_copy(outbuf, shared.at[sid])

    plsc.subcore_barrier()

    @pl.when(sid % 2 == 0)
    def _():
        pltpu.async_copy(shared.at[sid + 1], outbuf, sem_m).wait()
        outbuf[...] = acc + outbuf[...]
        b0 = base
        c1 = pltpu.async_copy(outbuf.at[pl.ds(0, 8)],
                              out_hbm.at[pl.ds(_al(b0, 8), 8)], sem_m)
        c2 = pltpu.async_copy(outbuf.at[pl.ds(8, 8)],
                              out_hbm.at[pl.ds(_al(B + b0, 8), 8)], sem_m)
        c1.wait()
        c2.wait()


def kernel(temporal_score, temporal_offset, frames_id):
    B, T = temporal_score.shape
    ML = frames_id.shape[1]
    info = plsc.get_sparse_core_info()
    NC, NS = info.num_cores, info.num_subcores
    NW = NC * NS
    RPW = B // NW

    score_po = temporal_score.reshape(B // 8, 8, T // 128, 128).transpose(
        0, 2, 1, 3)
    off_po = temporal_offset.reshape(B, T // 128, 128, 2).transpose(
        0, 1, 3, 2)
    frames_po = frames_id.reshape(B // 8, 8, ML // 128, 128).transpose(
        0, 2, 1, 3)

    mesh = plsc.VectorSubcoreMesh(core_axis_name="c", subcore_axis_name="s")
    k = functools.partial(
        pl.kernel,
        mesh=mesh,
        out_type=jax.ShapeDtypeStruct((2 * B,), jnp.float32),
        scratch_types=[
            pltpu.VMEM((RPW, T // 128, 128), jnp.float32),
            pltpu.VMEM((2 * RPW, _L), jnp.float32),
            pltpu.VMEM((2 * RPW, _L), jnp.int32),
            pltpu.VMEM((_L,), jnp.float32),
            pltpu.VMEM_SHARED((NS, _L), jnp.float32),
            [pltpu.SemaphoreType.DMA] * RPW,
            [pltpu.SemaphoreType.DMA] * RPW,
            pltpu.SemaphoreType.DMA,
            pltpu.SemaphoreType.DMA,
        ],
        compiler_params=pltpu.CompilerParams(
            use_tc_tiling_on_sc=False, needs_layout_passes=False
        ),
    )(functools.partial(_sc_kernel_body, NS, B, T, ML, RPW))
    return k(score_po, off_po, frames_po).reshape(2, B).T
